# baseline (speedup 1.0000x reference)
import functools

import jax
import jax.numpy as jnp
from jax import lax
from jax.experimental import pallas as pl
from jax.experimental.pallas import tpu as pltpu
from jax.experimental.pallas import tpu_sc as plsc

_ALPHA = 0.5


def _make_gather(C, D, B):
    NC, NS = 2, 16
    NW = NC * NS
    b_per_w = B // NW
    mesh = plsc.VectorSubcoreMesh(core_axis_name="c", subcore_axis_name="s")

    @functools.partial(
        pl.kernel,
        out_type=jax.ShapeDtypeStruct((B, D), jnp.float32),
        mesh=mesh,
        scratch_types=[
            pltpu.VMEM((b_per_w,), jnp.int32),
            pltpu.VMEM((b_per_w, D), jnp.float32),
            pltpu.SemaphoreType.DMA,
        ],
    )
    def gather_k(centers_hbm, idx_hbm, out_hbm, idx_v, rows_v, sem):
        wid = lax.axis_index("s") * NC + lax.axis_index("c")
        base = wid * b_per_w
        pltpu.sync_copy(idx_hbm.at[pl.ds(base, b_per_w)], idx_v)
        pltpu.async_copy(centers_hbm.at[idx_v], rows_v, sem).wait()
        pltpu.sync_copy(rows_v, out_hbm.at[pl.ds(base, b_per_w)])

    return gather_k


def _tc_body(H, lcol_ref, lrow_ref, f_blk_ref, cb_blk_ref, f_all_ref,
             cb_all_ref, sent_ref, res_ref, val0_ref, val1_ref):
    lcol = lcol_ref[...]
    lrow = lrow_ref[...]
    eqf = (lcol == lrow).astype(jnp.float32)
    appear = jnp.sum(eqf, axis=1, keepdims=True)
    d_all = cb_all_ref[...] - f_all_ref[...]
    m = jax.lax.dot_general(
        eqf, d_all, (((1,), (0,)), ((), ())),
        preferred_element_type=jnp.float32)
    scale = _ALPHA / (1.0 + appear)
    cb_blk = cb_blk_ref[...]
    u = cb_blk - scale * m
    r = f_blk_ref[...] - cb_blk
    res_ref[...] = jnp.sum(r * r, axis=1, keepdims=True)

    def sent_row(s, idx):
        mask = (lrow == s).astype(jnp.float32)
        n = jnp.sum(mask)
        v = jax.lax.dot_general(
            mask, d_all, (((1,), (0,)), ((), ())),
            preferred_element_type=jnp.float32)
        return sent_ref[idx, :].reshape(1, -1) - (_ALPHA / (1.0 + n)) * v

    row0 = sent_row(0, 0)
    rowh = sent_row(H, 1)
    in0 = lcol < H
    val0_ref[...] = jnp.where(in0, u, row0)
    val1_ref[...] = jnp.where(in0, rowh, u)


def _tc_math(labels, features, cb, sent_centers, H):
    B, D = features.shape
    BI = 512
    nblk = B // BI
    lcol = labels.reshape(B, 1)
    lrow = labels.reshape(1, B)
    return pl.pallas_call(
        functools.partial(_tc_body, H),
        grid=(nblk,),
        in_specs=[
            pl.BlockSpec((BI, 1), lambda i: (i, 0)),
            pl.BlockSpec((1, B), lambda i: (0, 0)),
            pl.BlockSpec((BI, D), lambda i: (i, 0)),
            pl.BlockSpec((BI, D), lambda i: (i, 0)),
            pl.BlockSpec((B, D), lambda i: (0, 0)),
            pl.BlockSpec((B, D), lambda i: (0, 0)),
            pl.BlockSpec((2, D), lambda i: (0, 0)),
        ],
        out_specs=[
            pl.BlockSpec((BI, 1), lambda i: (i, 0)),
            pl.BlockSpec((BI, D), lambda i: (i, 0)),
            pl.BlockSpec((BI, D), lambda i: (i, 0)),
        ],
        out_shape=[
            jax.ShapeDtypeStruct((B, 1), jnp.float32),
            jax.ShapeDtypeStruct((B, D), jnp.float32),
            jax.ShapeDtypeStruct((B, D), jnp.float32),
        ],
    )(lcol, lrow, features, cb, features, cb, sent_centers)


def _make_copy_scatter(C, D, B):
    NC, NS = 2, 16
    H = C // 2
    CC = 400
    nchunks = H // CC
    slots = -(-nchunks // NS)
    upd_per_w = B // NS
    CH = 128
    assert H % CC == 0 and CC % 8 == 0 and upd_per_w % CH == 0
    mesh = plsc.VectorSubcoreMesh(core_axis_name="c", subcore_axis_name="s")

    @functools.partial(
        pl.kernel,
        out_type=jax.ShapeDtypeStruct((C, D), jnp.float32),
        mesh=mesh,
        scratch_types=[
            pltpu.VMEM((CH,), jnp.int32),
            pltpu.VMEM((CH, D), jnp.float32),
            pltpu.SemaphoreType.DMA,
            pltpu.SemaphoreType.DMA,
        ],
    )
    def copy_scatter_k(centers_hbm, idx0_hbm, idx1_hbm, val0_hbm, val1_hbm,
                       out_hbm, idx_v, rows_v, csem, ssem):
        cid = lax.axis_index("c")
        sid = lax.axis_index("s")
        core_base = cid * H

        def chunk_base(j):
            return core_base + (sid + j * NS) * CC

        for j in range(slots):
            @pl.when(sid + j * NS < nchunks)
            def _():
                pltpu.async_copy(
                    centers_hbm.at[pl.ds(chunk_base(j), CC)],
                    out_hbm.at[pl.ds(chunk_base(j), CC)], csem)
        for j in range(slots):
            @pl.when(sid + j * NS < nchunks)
            def _():
                pltpu.make_async_copy(
                    centers_hbm.at[pl.ds(chunk_base(j), CC)],
                    out_hbm.at[pl.ds(chunk_base(j), CC)], csem).wait()

        plsc.subcore_barrier()

        for c in range(upd_per_w // CH):
            ub = sid * upd_per_w + c * CH

            @pl.when(cid == 0)
            def _():
                pltpu.sync_copy(idx0_hbm.at[pl.ds(ub, CH)], idx_v)
                pltpu.sync_copy(val0_hbm.at[pl.ds(ub, CH)], rows_v)
                pltpu.async_copy(rows_v, out_hbm.at[idx_v], ssem).wait()

            @pl.when(cid == 1)
            def _():
                pltpu.sync_copy(idx1_hbm.at[pl.ds(ub, CH)], idx_v)
                pltpu.sync_copy(val1_hbm.at[pl.ds(ub, CH)], rows_v)
                pltpu.async_copy(rows_v, out_hbm.at[idx_v], ssem).wait()

    return copy_scatter_k


def kernel(features, labels, centers):
    labels = labels.reshape(-1).astype(jnp.int32)
    features = features.astype(jnp.float32)
    B, D = features.shape
    C = centers.shape[0]
    H = C // 2

    cb = _make_gather(C, D, B)(centers, labels)
    sent_centers = jnp.concatenate(
        [lax.slice(centers, (0, 0), (1, D)),
         lax.slice(centers, (H, 0), (H + 1, D))], axis=0)
    result, val0, val1 = _tc_math(labels, features, cb, sent_centers, H)
    idx0 = jnp.where(labels < H, labels, 0).astype(jnp.int32)
    idx1 = jnp.where(labels >= H, labels, H).astype(jnp.int32)
    new_centers = _make_copy_scatter(C, D, B)(
        centers, idx0, idx1, val0, val1)
    return (result, new_centers)


# speedup vs baseline: 10.1191x; 10.1191x over previous
import functools

import jax
import jax.numpy as jnp
from jax import lax
from jax.experimental import pallas as pl
from jax.experimental.pallas import tpu as pltpu
from jax.experimental.pallas import tpu_sc as plsc

_ALPHA = 0.5


def _make_gather(C, D, B):
    NC, NS = 2, 16
    NW = NC * NS
    b_per_w = B // NW
    mesh = plsc.VectorSubcoreMesh(core_axis_name="c", subcore_axis_name="s")

    @functools.partial(
        pl.kernel,
        out_type=jax.ShapeDtypeStruct((B, D), jnp.float32),
        mesh=mesh,
        scratch_types=[
            pltpu.VMEM((b_per_w,), jnp.int32),
            pltpu.VMEM((b_per_w, D), jnp.float32),
            pltpu.SemaphoreType.DMA,
        ],
    )
    def gather_k(centers_hbm, idx_hbm, out_hbm, idx_v, rows_v, sem):
        wid = lax.axis_index("s") * NC + lax.axis_index("c")
        base = wid * b_per_w
        pltpu.sync_copy(idx_hbm.at[pl.ds(base, b_per_w)], idx_v)
        pltpu.async_copy(centers_hbm.at[idx_v], rows_v, sem).wait()
        pltpu.sync_copy(rows_v, out_hbm.at[pl.ds(base, b_per_w)])

    return gather_k


def _tc_body(H, lcol_ref, lrow_ref, f_blk_ref, cb_blk_ref, f_all_ref,
             cb_all_ref, sent_ref, res_ref, val0_ref, val1_ref):
    lcol = lcol_ref[...]
    lrow = lrow_ref[...]
    eqf = (lcol == lrow).astype(jnp.float32)
    appear = jnp.sum(eqf, axis=1, keepdims=True)
    d_all = cb_all_ref[...] - f_all_ref[...]
    m = jax.lax.dot_general(
        eqf, d_all, (((1,), (0,)), ((), ())),
        preferred_element_type=jnp.float32)
    scale = _ALPHA / (1.0 + appear)
    cb_blk = cb_blk_ref[...]
    u = cb_blk - scale * m
    r = f_blk_ref[...] - cb_blk
    res_ref[...] = jnp.sum(r * r, axis=1, keepdims=True)

    def sent_row(s, idx):
        mask = (lrow == s).astype(jnp.float32)
        n = jnp.sum(mask)
        v = jax.lax.dot_general(
            mask, d_all, (((1,), (0,)), ((), ())),
            preferred_element_type=jnp.float32)
        return sent_ref[idx, :].reshape(1, -1) - (_ALPHA / (1.0 + n)) * v

    row0 = sent_row(0, 0)
    rowh = sent_row(H, 1)
    in0 = lcol < H
    val0_ref[...] = jnp.where(in0, u, row0)
    val1_ref[...] = jnp.where(in0, rowh, u)


def _tc_math(labels, features, cb, sent_centers, H):
    B, D = features.shape
    BI = 512
    nblk = B // BI
    lcol = labels.reshape(B, 1)
    lrow = labels.reshape(1, B)
    return pl.pallas_call(
        functools.partial(_tc_body, H),
        grid=(nblk,),
        in_specs=[
            pl.BlockSpec((BI, 1), lambda i: (i, 0)),
            pl.BlockSpec((1, B), lambda i: (0, 0)),
            pl.BlockSpec((BI, D), lambda i: (i, 0)),
            pl.BlockSpec((BI, D), lambda i: (i, 0)),
            pl.BlockSpec((B, D), lambda i: (0, 0)),
            pl.BlockSpec((B, D), lambda i: (0, 0)),
            pl.BlockSpec((2, D), lambda i: (0, 0)),
        ],
        out_specs=[
            pl.BlockSpec((BI, 1), lambda i: (i, 0)),
            pl.BlockSpec((BI, D), lambda i: (i, 0)),
            pl.BlockSpec((BI, D), lambda i: (i, 0)),
        ],
        out_shape=[
            jax.ShapeDtypeStruct((B, 1), jnp.float32),
            jax.ShapeDtypeStruct((B, D), jnp.float32),
            jax.ShapeDtypeStruct((B, D), jnp.float32),
        ],
    )(lcol, lrow, features, cb, features, cb, sent_centers)


def _make_copy_scatter(C, D, B):
    NC, NS = 2, 16
    H = C // 2
    CC = 200
    NBUF = 4
    nchunks = H // CC
    slots = -(-nchunks // NS)
    upd_per_w = B // NS
    CH = 128
    assert H % CC == 0 and CC % 8 == 0 and upd_per_w % CH == 0
    mesh = plsc.VectorSubcoreMesh(core_axis_name="c", subcore_axis_name="s")

    @functools.partial(
        pl.kernel,
        out_type=jax.ShapeDtypeStruct((C, D), jnp.float32),
        mesh=mesh,
        scratch_types=(
            [pltpu.VMEM((CC, D), jnp.float32)] * NBUF
            + [pltpu.VMEM((CH,), jnp.int32), pltpu.VMEM((CH, D), jnp.float32)]
            + [pltpu.SemaphoreType.DMA] * (2 * NBUF + 1)
        ),
    )
    def copy_scatter_k(centers_hbm, idx0_hbm, idx1_hbm, val0_hbm, val1_hbm,
                       out_hbm, *scratch):
        bufs = scratch[:NBUF]
        idx_v, rows_v = scratch[NBUF], scratch[NBUF + 1]
        isems = scratch[NBUF + 2:2 * NBUF + 2]
        osems = scratch[2 * NBUF + 2:3 * NBUF + 2]
        ssem = scratch[3 * NBUF + 2]
        cid = lax.axis_index("c")
        sid = lax.axis_index("s")
        core_base = cid * H

        def chunk_base(j):
            return core_base + (sid + j * NS) * CC

        def start_in(j):
            @pl.when(sid + j * NS < nchunks)
            def _():
                pltpu.async_copy(
                    centers_hbm.at[pl.ds(chunk_base(j), CC)],
                    bufs[j % NBUF], isems[j % NBUF])

        def wait_in(j):
            @pl.when(sid + j * NS < nchunks)
            def _():
                pltpu.make_async_copy(
                    centers_hbm.at[pl.ds(chunk_base(j), CC)],
                    bufs[j % NBUF], isems[j % NBUF]).wait()

        def start_out(j):
            @pl.when(sid + j * NS < nchunks)
            def _():
                pltpu.async_copy(
                    bufs[j % NBUF],
                    out_hbm.at[pl.ds(chunk_base(j), CC)], osems[j % NBUF])

        def wait_out(j):
            @pl.when(sid + j * NS < nchunks)
            def _():
                pltpu.make_async_copy(
                    bufs[j % NBUF],
                    out_hbm.at[pl.ds(chunk_base(j), CC)], osems[j % NBUF]).wait()

        for j in range(NBUF - 1):
            start_in(j)
        for j in range(slots):
            if j >= NBUF - 1:
                wait_out(j - (NBUF - 1))
            if j + NBUF - 1 < slots:
                start_in(j + NBUF - 1)
            wait_in(j)
            start_out(j)
        for j in range(max(0, slots - (NBUF - 1)), slots):
            wait_out(j)

        plsc.subcore_barrier()

        for c in range(upd_per_w // CH):
            ub = sid * upd_per_w + c * CH

            @pl.when(cid == 0)
            def _():
                pltpu.sync_copy(idx0_hbm.at[pl.ds(ub, CH)], idx_v)
                pltpu.sync_copy(val0_hbm.at[pl.ds(ub, CH)], rows_v)
                pltpu.async_copy(rows_v, out_hbm.at[idx_v], ssem).wait()

            @pl.when(cid == 1)
            def _():
                pltpu.sync_copy(idx1_hbm.at[pl.ds(ub, CH)], idx_v)
                pltpu.sync_copy(val1_hbm.at[pl.ds(ub, CH)], rows_v)
                pltpu.async_copy(rows_v, out_hbm.at[idx_v], ssem).wait()

    return copy_scatter_k


def kernel(features, labels, centers):
    labels = labels.reshape(-1).astype(jnp.int32)
    features = features.astype(jnp.float32)
    B, D = features.shape
    C = centers.shape[0]
    H = C // 2

    cb = _make_gather(C, D, B)(centers, labels)
    sent_centers = jnp.concatenate(
        [lax.slice(centers, (0, 0), (1, D)),
         lax.slice(centers, (H, 0), (H + 1, D))], axis=0)
    result, val0, val1 = _tc_math(labels, features, cb, sent_centers, H)
    idx0 = jnp.where(labels < H, labels, 0).astype(jnp.int32)
    idx1 = jnp.where(labels >= H, labels, H).astype(jnp.int32)
    new_centers = _make_copy_scatter(C, D, B)(
        centers, idx0, idx1, val0, val1)
    return (result, new_centers)


# speedup vs baseline: 17.9105x; 1.7700x over previous
import functools

import jax
import jax.numpy as jnp
from jax import lax
from jax.experimental import pallas as pl
from jax.experimental.pallas import tpu as pltpu
from jax.experimental.pallas import tpu_sc as plsc

_ALPHA = 0.5


def _make_gather(C, D, B):
    NC, NS = 2, 16
    NW = NC * NS
    b_per_w = B // NW
    mesh = plsc.VectorSubcoreMesh(core_axis_name="c", subcore_axis_name="s")

    @functools.partial(
        pl.kernel,
        out_type=jax.ShapeDtypeStruct((B, D), jnp.float32),
        mesh=mesh,
        scratch_types=[
            pltpu.VMEM((b_per_w,), jnp.int32),
            pltpu.VMEM((b_per_w, D), jnp.float32),
            pltpu.SemaphoreType.DMA,
        ],
    )
    def gather_k(centers_hbm, idx_hbm, out_hbm, idx_v, rows_v, sem):
        wid = lax.axis_index("s") * NC + lax.axis_index("c")
        base = wid * b_per_w
        pltpu.sync_copy(idx_hbm.at[pl.ds(base, b_per_w)], idx_v)
        pltpu.async_copy(centers_hbm.at[idx_v], rows_v, sem).wait()
        pltpu.sync_copy(rows_v, out_hbm.at[pl.ds(base, b_per_w)])

    return gather_k


def _tc_body(lcol_ref, lrow_ref, f_blk_ref, cb_blk_ref, f_all_ref,
             cb_all_ref, res_ref, u_ref):
    lcol = lcol_ref[...]
    lrow = lrow_ref[...]
    eqf = (lcol == lrow).astype(jnp.float32)
    appear = jnp.sum(eqf, axis=1, keepdims=True)
    d_all = cb_all_ref[...] - f_all_ref[...]
    m = jax.lax.dot_general(
        eqf, d_all, (((1,), (0,)), ((), ())),
        preferred_element_type=jnp.float32)
    scale = _ALPHA / (1.0 + appear)
    cb_blk = cb_blk_ref[...]
    u_ref[...] = cb_blk - scale * m
    r = f_blk_ref[...] - cb_blk
    res_ref[...] = jnp.sum(r * r, axis=1, keepdims=True)


def _tc_math(labels, features, cb):
    B, D = features.shape
    BI = 512
    nblk = B // BI
    lcol = labels.reshape(B, 1)
    lrow = labels.reshape(1, B)
    return pl.pallas_call(
        _tc_body,
        grid=(nblk,),
        in_specs=[
            pl.BlockSpec((BI, 1), lambda i: (i, 0)),
            pl.BlockSpec((1, B), lambda i: (0, 0)),
            pl.BlockSpec((BI, D), lambda i: (i, 0)),
            pl.BlockSpec((BI, D), lambda i: (i, 0)),
            pl.BlockSpec((B, D), lambda i: (0, 0)),
            pl.BlockSpec((B, D), lambda i: (0, 0)),
        ],
        out_specs=[
            pl.BlockSpec((BI, 1), lambda i: (i, 0)),
            pl.BlockSpec((BI, D), lambda i: (i, 0)),
        ],
        out_shape=[
            jax.ShapeDtypeStruct((B, 1), jnp.float32),
            jax.ShapeDtypeStruct((B, D), jnp.float32),
        ],
    )(lcol, lrow, features, cb, features, cb)


def _copy_body(src_ref, dst_ref):
    dst_ref[...] = src_ref[...]


def _tc_copy(centers):
    C, D = centers.shape
    R = 2000
    assert C % R == 0
    return pl.pallas_call(
        _copy_body,
        grid=(C // R,),
        in_specs=[pl.BlockSpec((R, D), lambda i: (i, 0))],
        out_specs=pl.BlockSpec((R, D), lambda i: (i, 0)),
        out_shape=jax.ShapeDtypeStruct((C, D), jnp.float32),
    )(centers)


def _make_scatter(C, D, B):
    NC, NS = 2, 16
    NW = NC * NS
    b_per_w = B // NW
    mesh = plsc.VectorSubcoreMesh(core_axis_name="c", subcore_axis_name="s")

    @functools.partial(
        pl.kernel,
        out_type=jax.ShapeDtypeStruct((b_per_w,), jnp.int32),
        mesh=mesh,
        scratch_types=[
            pltpu.VMEM((b_per_w,), jnp.int32),
            pltpu.VMEM((b_per_w, D), jnp.float32),
            pltpu.SemaphoreType.DMA,
        ],
        compiler_params=pltpu.CompilerParams(has_side_effects=True),
    )
    def scatter_k(table_hbm, idx_hbm, val_hbm, tok_hbm, idx_v, rows_v, sem):
        cid = lax.axis_index("c")
        sid = lax.axis_index("s")
        wid = sid * NC + cid
        base = wid * b_per_w
        pltpu.sync_copy(idx_hbm.at[pl.ds(base, b_per_w)], idx_v)
        pltpu.sync_copy(val_hbm.at[pl.ds(base, b_per_w)], rows_v)
        pltpu.async_copy(rows_v, table_hbm.at[idx_v], sem).wait()

        @pl.when(wid == 0)
        def _():
            pltpu.sync_copy(idx_v, tok_hbm)

    return scatter_k


def kernel(features, labels, centers):
    labels = labels.reshape(-1).astype(jnp.int32)
    features = features.astype(jnp.float32)
    B, D = features.shape
    C = centers.shape[0]

    cb = _make_gather(C, D, B)(centers, labels)
    result, u = _tc_math(labels, features, cb)
    table = _tc_copy(centers)
    tok = _make_scatter(C, D, B)(table, labels, u)
    new_centers, _ = lax.optimization_barrier((table, tok))
    return (result, new_centers)


# speedup vs baseline: 18.0731x; 1.0091x over previous
import functools

import jax
import jax.numpy as jnp
from jax import lax
from jax.experimental import pallas as pl
from jax.experimental.pallas import tpu as pltpu
from jax.experimental.pallas import tpu_sc as plsc

_ALPHA = 0.5


def _make_gather(C, D, B):
    NC, NS = 2, 16
    NW = NC * NS
    b_per_w = B // NW
    mesh = plsc.VectorSubcoreMesh(core_axis_name="c", subcore_axis_name="s")

    @functools.partial(
        pl.kernel,
        out_type=jax.ShapeDtypeStruct((B, D), jnp.float32),
        mesh=mesh,
        scratch_types=[
            pltpu.VMEM((b_per_w,), jnp.int32),
            pltpu.VMEM((b_per_w, D), jnp.float32),
            pltpu.SemaphoreType.DMA,
        ],
    )
    def gather_k(centers_hbm, idx_hbm, out_hbm, idx_v, rows_v, sem):
        wid = lax.axis_index("s") * NC + lax.axis_index("c")
        base = wid * b_per_w
        pltpu.sync_copy(idx_hbm.at[pl.ds(base, b_per_w)], idx_v)
        pltpu.async_copy(centers_hbm.at[idx_v], rows_v, sem).wait()
        pltpu.sync_copy(rows_v, out_hbm.at[pl.ds(base, b_per_w)])

    return gather_k


_BI = 512


def _tc_body(lcol_ref, lrow_ref, f_ref, cb_ref, res_ref, u_ref):
    B, D = f_ref.shape
    lrow = lrow_ref[...]
    d_all = cb_ref[...] - f_ref[...]
    for i in range(B // _BI):
        sl = pl.ds(i * _BI, _BI)
        lcol = lcol_ref[sl, :]
        eqf = (lcol == lrow).astype(jnp.float32)
        appear = jnp.sum(eqf, axis=1, keepdims=True)
        m = jax.lax.dot_general(
            eqf, d_all, (((1,), (0,)), ((), ())),
            preferred_element_type=jnp.float32)
        scale = _ALPHA / (1.0 + appear)
        cb_blk = cb_ref[sl, :]
        u_ref[sl, :] = cb_blk - scale * m
        r = f_ref[sl, :] - cb_blk
        res_ref[sl, :] = jnp.sum(r * r, axis=1, keepdims=True)


def _tc_math(labels, features, cb):
    B, D = features.shape
    lcol = labels.reshape(B, 1)
    lrow = labels.reshape(1, B)
    return pl.pallas_call(
        _tc_body,
        in_specs=[
            pl.BlockSpec((B, 1), lambda: (0, 0)),
            pl.BlockSpec((1, B), lambda: (0, 0)),
            pl.BlockSpec((B, D), lambda: (0, 0)),
            pl.BlockSpec((B, D), lambda: (0, 0)),
        ],
        out_specs=[
            pl.BlockSpec((B, 1), lambda: (0, 0)),
            pl.BlockSpec((B, D), lambda: (0, 0)),
        ],
        out_shape=[
            jax.ShapeDtypeStruct((B, 1), jnp.float32),
            jax.ShapeDtypeStruct((B, D), jnp.float32),
        ],
    )(lcol, lrow, features, cb)


def _copy_body(src_ref, dst_ref):
    dst_ref[...] = src_ref[...]


def _tc_copy(centers):
    C, D = centers.shape
    R = 2000
    assert C % R == 0
    return pl.pallas_call(
        _copy_body,
        grid=(C // R,),
        in_specs=[pl.BlockSpec((R, D), lambda i: (i, 0))],
        out_specs=pl.BlockSpec((R, D), lambda i: (i, 0)),
        out_shape=jax.ShapeDtypeStruct((C, D), jnp.float32),
    )(centers)


def _make_scatter(C, D, B):
    NC, NS = 2, 16
    NW = NC * NS
    b_per_w = B // NW
    mesh = plsc.VectorSubcoreMesh(core_axis_name="c", subcore_axis_name="s")

    @functools.partial(
        pl.kernel,
        out_type=jax.ShapeDtypeStruct((b_per_w,), jnp.int32),
        mesh=mesh,
        scratch_types=[
            pltpu.VMEM((b_per_w,), jnp.int32),
            pltpu.VMEM((b_per_w, D), jnp.float32),
            pltpu.SemaphoreType.DMA,
        ],
        compiler_params=pltpu.CompilerParams(has_side_effects=True),
    )
    def scatter_k(table_hbm, idx_hbm, val_hbm, tok_hbm, idx_v, rows_v, sem):
        cid = lax.axis_index("c")
        sid = lax.axis_index("s")
        wid = sid * NC + cid
        base = wid * b_per_w
        pltpu.sync_copy(idx_hbm.at[pl.ds(base, b_per_w)], idx_v)
        pltpu.sync_copy(val_hbm.at[pl.ds(base, b_per_w)], rows_v)
        pltpu.async_copy(rows_v, table_hbm.at[idx_v], sem).wait()

        @pl.when(wid == 0)
        def _():
            pltpu.sync_copy(idx_v, tok_hbm)

    return scatter_k


def kernel(features, labels, centers):
    labels = labels.reshape(-1).astype(jnp.int32)
    features = features.astype(jnp.float32)
    B, D = features.shape
    C = centers.shape[0]

    cb = _make_gather(C, D, B)(centers, labels)
    result, u = _tc_math(labels, features, cb)
    table = _tc_copy(centers)
    tok = _make_scatter(C, D, B)(table, labels, u)
    new_centers, _ = lax.optimization_barrier((table, tok))
    return (result, new_centers)
